# baseline (speedup 1.0000x reference)
import jax
import jax.numpy as jnp
from jax.experimental import pallas as pl
from jax.experimental.pallas import tpu as pltpu

_NCHUNK = 20
_CHUNK = 500


def _sigmoid(v):
    return 0.5 * jnp.tanh(v * 0.5) + 0.5


def _sigmoid_manual(x_hbm, o_hbm, x_vmem, o_vmem, in_sems, out_sems):
    in_copies = []
    for c in range(_NCHUNK):
        sl = pl.ds(c * _CHUNK, _CHUNK)
        cp = pltpu.make_async_copy(
            x_hbm.at[sl, :], x_vmem.at[sl, :], in_sems.at[c]
        )
        cp.start()
        in_copies.append(cp)
    out_copies = []
    for c in range(_NCHUNK):
        sl = pl.ds(c * _CHUNK, _CHUNK)
        in_copies[c].wait()
        o_vmem[sl, :] = _sigmoid(x_vmem[sl, :])
        cp = pltpu.make_async_copy(
            o_vmem.at[sl, :], o_hbm.at[sl, :], out_sems.at[c]
        )
        cp.start()
        out_copies.append(cp)
    for cp in out_copies:
        cp.wait()


def kernel(x, edge_index):
    del edge_index
    n_rows, d = x.shape
    return pl.pallas_call(
        _sigmoid_manual,
        in_specs=[pl.BlockSpec(memory_space=pltpu.MemorySpace.HBM)],
        out_specs=pl.BlockSpec(memory_space=pltpu.MemorySpace.HBM),
        out_shape=jax.ShapeDtypeStruct(x.shape, x.dtype),
        scratch_shapes=[
            pltpu.VMEM((n_rows, d), jnp.float32),
            pltpu.VMEM((n_rows, d), jnp.float32),
            pltpu.SemaphoreType.DMA((_NCHUNK,)),
            pltpu.SemaphoreType.DMA((_NCHUNK,)),
        ],
    )(x)


# speedup vs baseline: 1.0814x; 1.0814x over previous
import jax
import jax.numpy as jnp
from jax.experimental import pallas as pl
from jax.experimental.pallas import tpu as pltpu

_NCHUNK = 5
_CHUNK = 2000


def _sigmoid(v):
    return 0.5 * jnp.tanh(v * 0.5) + 0.5


def _sigmoid_manual(x_hbm, o_hbm, x_vmem, o_vmem, in_sems, out_sems):
    in_copies = []
    for c in range(_NCHUNK):
        sl = pl.ds(c * _CHUNK, _CHUNK)
        cp = pltpu.make_async_copy(
            x_hbm.at[sl, :], x_vmem.at[sl, :], in_sems.at[c]
        )
        cp.start()
        in_copies.append(cp)
    out_copies = []
    for c in range(_NCHUNK):
        sl = pl.ds(c * _CHUNK, _CHUNK)
        in_copies[c].wait()
        o_vmem[sl, :] = _sigmoid(x_vmem[sl, :])
        cp = pltpu.make_async_copy(
            o_vmem.at[sl, :], o_hbm.at[sl, :], out_sems.at[c]
        )
        cp.start()
        out_copies.append(cp)
    for cp in out_copies:
        cp.wait()


def kernel(x, edge_index):
    del edge_index
    n_rows, d = x.shape
    return pl.pallas_call(
        _sigmoid_manual,
        in_specs=[pl.BlockSpec(memory_space=pltpu.MemorySpace.HBM)],
        out_specs=pl.BlockSpec(memory_space=pltpu.MemorySpace.HBM),
        out_shape=jax.ShapeDtypeStruct(x.shape, x.dtype),
        scratch_shapes=[
            pltpu.VMEM((n_rows, d), jnp.float32),
            pltpu.VMEM((n_rows, d), jnp.float32),
            pltpu.SemaphoreType.DMA((_NCHUNK,)),
            pltpu.SemaphoreType.DMA((_NCHUNK,)),
        ],
    )(x)
